# baseline (speedup 1.0000x reference)
import functools

import jax
import jax.numpy as jnp
import numpy as np
from jax import lax
from jax.experimental import pallas as pl
from jax.experimental.pallas import tpu as pltpu
from jax.experimental.pallas import tpu_sc as plsc

VOCAB = 50257
NLAYER = 2
NHEAD = 12
NEMBD = 768
HD = NEMBD // NHEAD
NEXP = 8
T = 2048

TBLK = 512
VBLK = 512


def _sc_gather(table, idx):
    info = plsc.get_sparse_core_info()
    nw = info.num_cores * info.num_subcores
    bpw = T // nw
    mesh = plsc.VectorSubcoreMesh(core_axis_name="c", subcore_axis_name="s")

    @functools.partial(
        pl.kernel,
        mesh=mesh,
        out_type=jax.ShapeDtypeStruct((T, NEMBD), jnp.float32),
        scratch_types=[
            pltpu.VMEM((bpw,), jnp.int32),
            pltpu.VMEM((bpw, NEMBD), jnp.float32),
            pltpu.SemaphoreType.DMA,
        ],
    )
    def k(table_hbm, idx_hbm, out_hbm, idx_v, rows_v, sem):
        wid = lax.axis_index("s") * info.num_cores + lax.axis_index("c")
        base = wid * bpw
        pltpu.sync_copy(idx_hbm.at[pl.ds(base, bpw)], idx_v)
        pltpu.async_copy(table_hbm.at[idx_v], rows_v, sem).wait()
        pltpu.sync_copy(rows_v, out_hbm.at[pl.ds(base, bpw)])

    return k(table, idx)


def _ln(x, g, b):
    m = jnp.mean(x, -1, keepdims=True)
    v = jnp.var(x, -1, keepdims=True)
    return (x - m) / jnp.sqrt(v + 1e-5) * g + b


def _gelu(x):
    return 0.5 * x * (1.0 + jnp.tanh(np.sqrt(2.0 / np.pi) * (x + 0.044715 * x**3)))


def _softplus(x):
    return jnp.maximum(x, 0.0) + jnp.log1p(jnp.exp(-jnp.abs(x)))


def _dot_t(a, b):
    return lax.dot_general(a, b, (((1,), (1,)), ((), ())),
                           preferred_element_type=jnp.float32)


def _layer_body(has_xb, last, *refs):
    if has_xb:
        (xa, xb, xa0, xb0, eps, ln1g, ln1b, wvp, wof, gatew, noisew,
         ln2g, ln2b, fcw, fcb, pjw, pjb, *rest) = refs
    else:
        (xa, xa0, eps, ln1g, ln1b, wvp, wof, gatew, noisew,
         ln2g, ln2b, fcw, fcb, pjw, pjb, *rest) = refs
        xb = xb0 = None
    if last:
        lnfg, lnfb, out, osc = rest
    else:
        out, osc = rest

    x = xa[...] + xb[...] if has_xb else xa[...]
    g1, b1 = ln1g[...], ln1b[...]
    xs = _ln(x, g1, b1)

    @pl.when(pl.program_id(0) == 0)
    def _():
        row0 = xa0[0:1, :] + xb0[0:1, :] if has_xb else xa0[0:1, :]
        xs0 = _ln(row0, g1, b1)
        attn = _dot_t(xs0, wvp[...])
        for i in range(NEXP):
            osc[i:i + 1, :] = _dot_t(attn, wof[i * NEMBD:(i + 1) * NEMBD, :])

    g = _dot_t(xs, gatew[...])
    ns = _softplus(_dot_t(xs, noisew[...]))
    g = g + eps[...] * ns
    m1 = jnp.max(g, axis=1, keepdims=True)
    m2 = jnp.max(jnp.where(g >= m1, jnp.float32(-1e30), g), axis=1, keepdims=True)
    denom = 1.0 + jnp.exp(m2 - m1)
    coef = jnp.where(g >= m2, jnp.exp(g - m1) / denom, 0.0)

    moa = coef[:, 0:1] * osc[0:1, :]
    for i in range(1, NEXP):
        moa = moa + coef[:, i:i + 1] * osc[i:i + 1, :]

    x1 = x + moa
    h = _gelu(_dot_t(_ln(x1, ln2g[...], ln2b[...]), fcw[...]) + fcb[...])
    y = x1 + _dot_t(h, pjw[...]) + pjb[...]
    if last:
        y = _ln(y, lnfg[...], lnfb[...])
    out[...] = y


def _layer_call(xa, xb, blk, eps, lnf):
    has_xb = xb is not None
    last = lnf is not None
    nsteps = T // TBLK

    def row2d(w):
        return w.reshape(1, -1)

    wvp = blk['Wv'].reshape(NHEAD, HD, NEMBD).transpose(1, 0, 2).reshape(NEMBD, NEMBD)
    wof = blk['Wo'].reshape(NEXP * NEMBD, NEMBD)

    full = lambda shape: pl.BlockSpec(shape, lambda i: (0,) * len(shape))
    tok = lambda shape: pl.BlockSpec(shape, lambda i: (i,) + (0,) * (len(shape) - 1))

    args = [xa]
    specs = [tok((TBLK, NEMBD))]
    if has_xb:
        args += [xb]
        specs += [tok((TBLK, NEMBD))]
    args += [xa]
    specs += [full((8, NEMBD))]
    if has_xb:
        args += [xb]
        specs += [full((8, NEMBD))]
    args += [eps, row2d(blk['ln1_g']), row2d(blk['ln1_b']), wvp, wof,
             blk['gate'], blk['noise'], row2d(blk['ln2_g']), row2d(blk['ln2_b']),
             blk['fc_w'], row2d(blk['fc_b']), blk['proj_w'], row2d(blk['proj_b'])]
    specs += [tok((TBLK, NEXP)), full((1, NEMBD)), full((1, NEMBD)),
              full((NEMBD, NEMBD)), full((NEXP * NEMBD, NEMBD)),
              full((NEXP, NEMBD)), full((NEXP, NEMBD)),
              full((1, NEMBD)), full((1, NEMBD)),
              full((4 * NEMBD, NEMBD)), full((1, 4 * NEMBD)),
              full((NEMBD, 4 * NEMBD)), full((1, NEMBD))]
    if last:
        args += [row2d(lnf[0]), row2d(lnf[1])]
        specs += [full((1, NEMBD)), full((1, NEMBD))]

    return pl.pallas_call(
        functools.partial(_layer_body, has_xb, last),
        grid=(nsteps,),
        in_specs=specs,
        out_specs=tok((TBLK, NEMBD)),
        out_shape=jax.ShapeDtypeStruct((T, NEMBD), jnp.float32),
        scratch_shapes=[pltpu.VMEM((NEXP, NEMBD), jnp.float32)],
    )(*args)


def _logits_body(xf_ref, wte_ref, out_ref):
    r = lax.dot_general(wte_ref[...], xf_ref[...], (((1,), (1,)), ((), ())),
                        preferred_element_type=jnp.float32)
    out_ref[...] = r[:, None, :]


def _logits_call(xf_bf16, wte):
    nsteps = pl.cdiv(VOCAB, VBLK)
    return pl.pallas_call(
        _logits_body,
        grid=(nsteps,),
        in_specs=[pl.BlockSpec((T, NEMBD), lambda i: (0, 0)),
                  pl.BlockSpec((VBLK, NEMBD), lambda i: (i, 0))],
        out_specs=pl.BlockSpec((VBLK, 1, T), lambda i: (i, 0, 0)),
        out_shape=jax.ShapeDtypeStruct((VOCAB, 1, T), jnp.float32),
    )(xf_bf16, wte)


def kernel(params, idx):
    p = params
    wte = p['wte']
    idxf = idx.reshape(T).astype(jnp.int32)
    tok = _sc_gather(wte, idxf)

    x = tok
    xb = p['wpe'][:T]
    for l, blk in enumerate(p['blocks']):
        eps = jax.random.normal(jax.random.fold_in(jax.random.key(42), l),
                                (T, NEXP), dtype=jnp.float32)
        last = l == NLAYER - 1
        lnf = (p['lnf_g'], p['lnf_b']) if last else None
        x = _layer_call(x, xb, blk, eps, lnf)
        xb = None

    logits_t = _logits_call(x, wte)
    return jnp.transpose(logits_t, (1, 2, 0))


# speedup vs baseline: 1.0601x; 1.0601x over previous
import functools

import jax
import jax.numpy as jnp
import numpy as np
from jax import lax
from jax.experimental import pallas as pl
from jax.experimental.pallas import tpu as pltpu
from jax.experimental.pallas import tpu_sc as plsc

VOCAB = 50257
NLAYER = 2
NHEAD = 12
NEMBD = 768
HD = NEMBD // NHEAD
NEXP = 8
T = 2048

TBLK = 512
VBLK = 1024


def _sc_gather(table, idx):
    info = plsc.get_sparse_core_info()
    nw = info.num_cores * info.num_subcores
    bpw = T // nw
    mesh = plsc.VectorSubcoreMesh(core_axis_name="c", subcore_axis_name="s")

    @functools.partial(
        pl.kernel,
        mesh=mesh,
        out_type=jax.ShapeDtypeStruct((T, NEMBD), jnp.float32),
        scratch_types=[
            pltpu.VMEM((bpw,), jnp.int32),
            pltpu.VMEM((bpw, NEMBD), jnp.float32),
            pltpu.SemaphoreType.DMA,
        ],
    )
    def k(table_hbm, idx_hbm, out_hbm, idx_v, rows_v, sem):
        wid = lax.axis_index("s") * info.num_cores + lax.axis_index("c")
        base = wid * bpw
        pltpu.sync_copy(idx_hbm.at[pl.ds(base, bpw)], idx_v)
        pltpu.async_copy(table_hbm.at[idx_v], rows_v, sem).wait()
        pltpu.sync_copy(rows_v, out_hbm.at[pl.ds(base, bpw)])

    return k(table, idx)


def _ln(x, g, b):
    m = jnp.mean(x, -1, keepdims=True)
    v = jnp.var(x, -1, keepdims=True)
    return (x - m) / jnp.sqrt(v + 1e-5) * g + b


def _gelu(x):
    return 0.5 * x * (1.0 + jnp.tanh(np.sqrt(2.0 / np.pi) * (x + 0.044715 * x**3)))


def _softplus(x):
    return jnp.maximum(x, 0.0) + jnp.log1p(jnp.exp(-jnp.abs(x)))


def _dot_t(a, b):
    return lax.dot_general(a, b, (((1,), (1,)), ((), ())),
                           preferred_element_type=jnp.float32)


def _layer_body(has_xb, last, *refs):
    if has_xb:
        (xa, xb, xa0, xb0, eps, ln1g, ln1b, wvp, wof, gatew, noisew,
         ln2g, ln2b, fcw, fcb, pjw, pjb, *rest) = refs
    else:
        (xa, xa0, eps, ln1g, ln1b, wvp, wof, gatew, noisew,
         ln2g, ln2b, fcw, fcb, pjw, pjb, *rest) = refs
        xb = xb0 = None
    if last:
        lnfg, lnfb, out, osc = rest
    else:
        out, osc = rest

    x = xa[...] + xb[...] if has_xb else xa[...]
    g1, b1 = ln1g[...], ln1b[...]
    xs = _ln(x, g1, b1)

    @pl.when(pl.program_id(0) == 0)
    def _():
        row0 = xa0[0:1, :] + xb0[0:1, :] if has_xb else xa0[0:1, :]
        xs0 = _ln(row0, g1, b1)
        attn = _dot_t(xs0, wvp[...])
        for i in range(NEXP):
            osc[i:i + 1, :] = _dot_t(attn, wof[i * NEMBD:(i + 1) * NEMBD, :])

    g = _dot_t(xs, gatew[...])
    ns = _softplus(_dot_t(xs, noisew[...]))
    g = g + eps[...] * ns
    m1 = jnp.max(g, axis=1, keepdims=True)
    m2 = jnp.max(jnp.where(g >= m1, jnp.float32(-1e30), g), axis=1, keepdims=True)
    denom = 1.0 + jnp.exp(m2 - m1)
    coef = jnp.where(g >= m2, jnp.exp(g - m1) / denom, 0.0)

    moa = coef[:, 0:1] * osc[0:1, :]
    for i in range(1, NEXP):
        moa = moa + coef[:, i:i + 1] * osc[i:i + 1, :]

    x1 = x + moa
    if last:
        bf = jnp.bfloat16
        h = _gelu(_dot_t(_ln(x1, ln2g[...], ln2b[...]).astype(bf),
                         fcw[...].astype(bf)) + fcb[...])
        y = x1 + _dot_t(h.astype(bf), pjw[...].astype(bf)) + pjb[...]
        y = _ln(y, lnfg[...], lnfb[...])
    else:
        h = _gelu(_dot_t(_ln(x1, ln2g[...], ln2b[...]), fcw[...]) + fcb[...])
        y = x1 + _dot_t(h, pjw[...]) + pjb[...]
    out[...] = y


def _layer_call(xa, xb, blk, eps, lnf):
    has_xb = xb is not None
    last = lnf is not None
    nsteps = T // TBLK

    def row2d(w):
        return w.reshape(1, -1)

    wvp = blk['Wv'].reshape(NHEAD, HD, NEMBD).transpose(1, 0, 2).reshape(NEMBD, NEMBD)
    wof = blk['Wo'].reshape(NEXP * NEMBD, NEMBD)

    full = lambda shape: pl.BlockSpec(shape, lambda i: (0,) * len(shape))
    tok = lambda shape: pl.BlockSpec(shape, lambda i: (i,) + (0,) * (len(shape) - 1))

    args = [xa]
    specs = [tok((TBLK, NEMBD))]
    if has_xb:
        args += [xb]
        specs += [tok((TBLK, NEMBD))]
    args += [xa]
    specs += [full((8, NEMBD))]
    if has_xb:
        args += [xb]
        specs += [full((8, NEMBD))]
    args += [eps, row2d(blk['ln1_g']), row2d(blk['ln1_b']), wvp, wof,
             blk['gate'], blk['noise'], row2d(blk['ln2_g']), row2d(blk['ln2_b']),
             blk['fc_w'], row2d(blk['fc_b']), blk['proj_w'], row2d(blk['proj_b'])]
    specs += [tok((TBLK, NEXP)), full((1, NEMBD)), full((1, NEMBD)),
              full((NEMBD, NEMBD)), full((NEXP * NEMBD, NEMBD)),
              full((NEXP, NEMBD)), full((NEXP, NEMBD)),
              full((1, NEMBD)), full((1, NEMBD)),
              full((4 * NEMBD, NEMBD)), full((1, 4 * NEMBD)),
              full((NEMBD, 4 * NEMBD)), full((1, NEMBD))]
    if last:
        args += [row2d(lnf[0]), row2d(lnf[1])]
        specs += [full((1, NEMBD)), full((1, NEMBD))]

    return pl.pallas_call(
        functools.partial(_layer_body, has_xb, last),
        grid=(nsteps,),
        in_specs=specs,
        out_specs=tok((TBLK, NEMBD)),
        out_shape=jax.ShapeDtypeStruct((T, NEMBD), jnp.float32),
        scratch_shapes=[pltpu.VMEM((NEXP, NEMBD), jnp.float32)],
    )(*args)


def _logits_body(xf_ref, wte_ref, out_ref):
    r = lax.dot_general(wte_ref[...], xf_ref[...], (((1,), (1,)), ((), ())),
                        preferred_element_type=jnp.float32)
    out_ref[...] = r[:, None, :]


def _logits_call(xf_bf16, wte):
    nsteps = pl.cdiv(VOCAB, VBLK)
    return pl.pallas_call(
        _logits_body,
        grid=(nsteps,),
        in_specs=[pl.BlockSpec((T, NEMBD), lambda i: (0, 0)),
                  pl.BlockSpec((VBLK, NEMBD), lambda i: (i, 0))],
        out_specs=pl.BlockSpec((VBLK, 1, T), lambda i: (i, 0, 0)),
        out_shape=jax.ShapeDtypeStruct((VOCAB, 1, T), jnp.float32),
    )(xf_bf16, wte)


def kernel(params, idx):
    p = params
    wte = p['wte']
    idxf = idx.reshape(T).astype(jnp.int32)
    tok = _sc_gather(wte, idxf)

    x = tok
    xb = p['wpe'][:T]
    for l, blk in enumerate(p['blocks']):
        eps = jax.random.normal(jax.random.fold_in(jax.random.key(42), l),
                                (T, NEXP), dtype=jnp.float32)
        last = l == NLAYER - 1
        lnf = (p['lnf_g'], p['lnf_b']) if last else None
        x = _layer_call(x, xb, blk, eps, lnf)
        xb = None

    logits_t = _logits_call(x, wte)
    return jnp.transpose(logits_t, (1, 2, 0))


# speedup vs baseline: 1.0652x; 1.0048x over previous
import functools

import jax
import jax.numpy as jnp
import numpy as np
from jax import lax
from jax.experimental import pallas as pl
from jax.experimental.pallas import tpu as pltpu
from jax.experimental.pallas import tpu_sc as plsc

VOCAB = 50257
NLAYER = 2
NHEAD = 12
NEMBD = 768
HD = NEMBD // NHEAD
NEXP = 8
T = 2048

TBLK = 512
VBLK = 1024


def _sc_gather(table, idx):
    info = plsc.get_sparse_core_info()
    nw = info.num_cores * info.num_subcores
    bpw = T // nw
    mesh = plsc.VectorSubcoreMesh(core_axis_name="c", subcore_axis_name="s")

    @functools.partial(
        pl.kernel,
        mesh=mesh,
        out_type=jax.ShapeDtypeStruct((T, NEMBD), jnp.float32),
        scratch_types=[
            pltpu.VMEM((bpw,), jnp.int32),
            pltpu.VMEM((bpw, NEMBD), jnp.float32),
            pltpu.SemaphoreType.DMA,
        ],
    )
    def k(table_hbm, idx_hbm, out_hbm, idx_v, rows_v, sem):
        wid = lax.axis_index("s") * info.num_cores + lax.axis_index("c")
        base = wid * bpw
        pltpu.sync_copy(idx_hbm.at[pl.ds(base, bpw)], idx_v)
        pltpu.async_copy(table_hbm.at[idx_v], rows_v, sem).wait()
        pltpu.sync_copy(rows_v, out_hbm.at[pl.ds(base, bpw)])

    return k(table, idx)


def _ln(x, g, b):
    m = jnp.mean(x, -1, keepdims=True)
    v = jnp.var(x, -1, keepdims=True)
    return (x - m) / jnp.sqrt(v + 1e-5) * g + b


def _gelu(x):
    return 0.5 * x * (1.0 + jnp.tanh(np.sqrt(2.0 / np.pi) * (x + 0.044715 * x**3)))


def _softplus(x):
    return jnp.maximum(x, 0.0) + jnp.log1p(jnp.exp(-jnp.abs(x)))


def _dot_t(a, b):
    return lax.dot_general(a, b, (((1,), (1,)), ((), ())),
                           preferred_element_type=jnp.float32)


def _layer_body(has_xb, last, *refs):
    if has_xb:
        (xa, xb, xa0, xb0, eps, ln1g, ln1b, wvp, wof, gatew, noisew,
         ln2g, ln2b, fcw, fcb, pjw, pjb, *rest) = refs
    else:
        (xa, xa0, eps, ln1g, ln1b, wvp, wof, gatew, noisew,
         ln2g, ln2b, fcw, fcb, pjw, pjb, *rest) = refs
        xb = xb0 = None
    if last:
        lnfg, lnfb, out, osc = rest
    else:
        out, osc = rest

    x = xa[...] + xb[...] if has_xb else xa[...]
    g1, b1 = ln1g[...], ln1b[...]
    xs = _ln(x, g1, b1)

    @pl.when(pl.program_id(0) == 0)
    def _():
        row0 = xa0[0:1, :] + xb0[0:1, :] if has_xb else xa0[0:1, :]
        xs0 = _ln(row0, g1, b1)
        attn = _dot_t(xs0, wvp[...])
        for i in range(NEXP):
            osc[i:i + 1, :] = _dot_t(attn, wof[i * NEMBD:(i + 1) * NEMBD, :])

    g = _dot_t(xs, gatew[...])
    ns = _softplus(_dot_t(xs, noisew[...]))
    g = g + eps[...] * ns
    m1 = jnp.max(g, axis=1, keepdims=True)
    m2 = jnp.max(jnp.where(g >= m1, jnp.float32(-1e30), g), axis=1, keepdims=True)
    denom = 1.0 + jnp.exp(m2 - m1)
    coef = jnp.where(g >= m2, jnp.exp(g - m1) / denom, 0.0)

    moa = coef[:, 0:1] * osc[0:1, :]
    for i in range(1, NEXP):
        moa = moa + coef[:, i:i + 1] * osc[i:i + 1, :]

    x1 = x + moa
    h = _gelu(_dot_t(_ln(x1, ln2g[...], ln2b[...]), fcw[...]) + fcb[...])
    y = x1 + _dot_t(h, pjw[...]) + pjb[...]
    if last:
        y = _ln(y, lnfg[...], lnfb[...])
    out[...] = y


def _layer_call(xa, xb, blk, eps, lnf):
    has_xb = xb is not None
    last = lnf is not None
    nsteps = T // TBLK

    def row2d(w):
        return w.reshape(1, -1)

    wvp = blk['Wv'].reshape(NHEAD, HD, NEMBD).transpose(1, 0, 2).reshape(NEMBD, NEMBD)
    wof = blk['Wo'].reshape(NEXP * NEMBD, NEMBD)

    full = lambda shape: pl.BlockSpec(shape, lambda i: (0,) * len(shape))
    tok = lambda shape: pl.BlockSpec(shape, lambda i: (i,) + (0,) * (len(shape) - 1))

    args = [xa]
    specs = [tok((TBLK, NEMBD))]
    if has_xb:
        args += [xb]
        specs += [tok((TBLK, NEMBD))]
    args += [xa]
    specs += [full((8, NEMBD))]
    if has_xb:
        args += [xb]
        specs += [full((8, NEMBD))]
    args += [eps, row2d(blk['ln1_g']), row2d(blk['ln1_b']), wvp, wof,
             blk['gate'], blk['noise'], row2d(blk['ln2_g']), row2d(blk['ln2_b']),
             blk['fc_w'], row2d(blk['fc_b']), blk['proj_w'], row2d(blk['proj_b'])]
    specs += [tok((TBLK, NEXP)), full((1, NEMBD)), full((1, NEMBD)),
              full((NEMBD, NEMBD)), full((NEXP * NEMBD, NEMBD)),
              full((NEXP, NEMBD)), full((NEXP, NEMBD)),
              full((1, NEMBD)), full((1, NEMBD)),
              full((4 * NEMBD, NEMBD)), full((1, 4 * NEMBD)),
              full((NEMBD, 4 * NEMBD)), full((1, NEMBD))]
    if last:
        args += [row2d(lnf[0]), row2d(lnf[1])]
        specs += [full((1, NEMBD)), full((1, NEMBD))]

    return pl.pallas_call(
        functools.partial(_layer_body, has_xb, last),
        grid=(nsteps,),
        in_specs=specs,
        out_specs=tok((TBLK, NEMBD)),
        out_shape=jax.ShapeDtypeStruct((T, NEMBD), jnp.float32),
        scratch_shapes=[pltpu.VMEM((NEXP, NEMBD), jnp.float32)],
    )(*args)


def _logits_body(xf_ref, wte_ref, out_ref):
    r = lax.dot_general(wte_ref[...], xf_ref[...], (((1,), (1,)), ((), ())),
                        preferred_element_type=jnp.float32)
    out_ref[...] = r[:, None, :]


def _logits_call(xf_bf16, wte):
    nsteps = pl.cdiv(VOCAB, VBLK)
    return pl.pallas_call(
        _logits_body,
        grid=(nsteps,),
        in_specs=[pl.BlockSpec((T, NEMBD), lambda i: (0, 0)),
                  pl.BlockSpec((VBLK, NEMBD), lambda i: (i, 0))],
        out_specs=pl.BlockSpec((VBLK, 1, T), lambda i: (i, 0, 0)),
        out_shape=jax.ShapeDtypeStruct((VOCAB, 1, T), jnp.float32),
    )(xf_bf16, wte)


def kernel(params, idx):
    p = params
    wte = p['wte']
    idxf = idx.reshape(T).astype(jnp.int32)
    tok = _sc_gather(wte, idxf)

    x = tok
    xb = p['wpe'][:T]
    for l, blk in enumerate(p['blocks']):
        eps = jax.random.normal(jax.random.fold_in(jax.random.key(42), l),
                                (T, NEXP), dtype=jnp.float32)
        last = l == NLAYER - 1
        lnf = (p['lnf_g'], p['lnf_b']) if last else None
        x = _layer_call(x, xb, blk, eps, lnf)
        xb = None

    logits_t = _logits_call(x, wte)
    return jnp.transpose(logits_t, (1, 2, 0))
